# baseline (speedup 1.0000x reference)
import functools

import jax
import jax.numpy as jnp
from jax import lax
from jax.experimental import pallas as pl
from jax.experimental.pallas import tpu as pltpu
from jax.experimental.pallas import tpu_sc as plsc


def _tc_body(x_ref, p_ref, o_ref):
    o_ref[...] = x_ref[...] + p_ref[...]


def _tc_kernel(x, pos_table):
    B, L, D = x.shape
    BL = 2048
    grid = (L // BL, B)
    return pl.pallas_call(
        _tc_body,
        grid=grid,
        in_specs=[
            pl.BlockSpec((1, BL, D), lambda l, b: (b, l, 0)),
            pl.BlockSpec((BL, D), lambda l, b: (l, 0)),
        ],
        out_specs=pl.BlockSpec((1, BL, D), lambda l, b: (b, l, 0)),
        out_shape=jax.ShapeDtypeStruct((B, L, D), x.dtype),
    )(x, pos_table)


_NC, _NS = 2, 16
_NW = _NC * _NS


def _make_sc_kernel(B, L, D):
    l_per_w = L // _NW
    R = 16
    CH = l_per_w // R
    W = R * D
    NV = W // 16
    mesh = plsc.VectorSubcoreMesh(core_axis_name="c", subcore_axis_name="s")

    @functools.partial(
        pl.kernel,
        mesh=mesh,
        out_type=jax.ShapeDtypeStruct((B * L * D,), jnp.float32),
        scratch_types=[
            pltpu.VMEM((B * W,), jnp.float32),
            pltpu.VMEM((W,), jnp.float32),
        ],
    )
    def k(x_hbm, pos_hbm, out_hbm, xbuf, pbuf):
        wid = lax.axis_index("s") * _NC + lax.axis_index("c")
        l0 = wid * l_per_w

        def chunk_body(c, carry):
            lbase = (l0 + c * R) * D
            pltpu.sync_copy(pos_hbm.at[pl.ds(lbase, W)], pbuf)
            for b in range(B):
                pltpu.sync_copy(x_hbm.at[pl.ds(b * L * D + lbase, W)],
                                xbuf.at[pl.ds(b * W, W)])

            def add_body(i, carry2):
                off = i * 16
                pv = pbuf[pl.ds(off, 16)]
                for b in range(B):
                    o = b * W + off
                    xbuf[pl.ds(o, 16)] = xbuf[pl.ds(o, 16)] + pv
                return carry2

            lax.fori_loop(0, NV, add_body, 0)
            for b in range(B):
                pltpu.sync_copy(xbuf.at[pl.ds(b * W, W)],
                                out_hbm.at[pl.ds(b * L * D + lbase, W)])
            return carry

        lax.fori_loop(0, CH, chunk_body, 0)

    return k


def _sc_kernel(x, pos_table):
    B, L, D = x.shape
    k = _make_sc_kernel(B, L, D)
    out = k(x.reshape(-1), pos_table[:L].reshape(-1))
    return out.reshape(B, L, D)


def _make_sc_kernel_v2(B, L, D):
    l_per_w = L // _NW
    R = 4
    CH = l_per_w // R
    W = R * D
    NV = W // 16
    U = 2
    mesh = plsc.VectorSubcoreMesh(core_axis_name="c", subcore_axis_name="s")

    @functools.partial(
        pl.kernel,
        mesh=mesh,
        out_type=jax.ShapeDtypeStruct((B * L * D,), jnp.float32),
        scratch_types=[
            pltpu.VMEM((B * W,), jnp.float32), pltpu.VMEM((B * W,), jnp.float32),
            pltpu.VMEM((W,), jnp.float32), pltpu.VMEM((W,), jnp.float32),
            pltpu.VMEM((B * W,), jnp.float32), pltpu.VMEM((B * W,), jnp.float32),
            pltpu.SemaphoreType.DMA, pltpu.SemaphoreType.DMA,
            pltpu.SemaphoreType.DMA, pltpu.SemaphoreType.DMA,
        ],
    )
    def k(x_hbm, pos_hbm, out_hbm, xb0, xb1, pb0, pb1, ob0, ob1,
          ls0, ls1, ss0, ss1):
        xb, pb, ob = (xb0, xb1), (pb0, pb1), (ob0, ob1)
        lsem, ssem = (ls0, ls1), (ss0, ss1)
        wid = lax.axis_index("s") * _NC + lax.axis_index("c")
        l0 = wid * l_per_w

        def start_load(c):
            p = c % 2
            lbase = (l0 + c * R) * D
            hs = [pltpu.async_copy(pos_hbm.at[pl.ds(lbase, W)], pb[p], lsem[p])]
            for b in range(B):
                hs.append(pltpu.async_copy(
                    x_hbm.at[pl.ds(b * L * D + lbase, W)],
                    xb[p].at[pl.ds(b * W, W)], lsem[p]))
            return hs

        def start_store(c):
            p = c % 2
            lbase = (l0 + c * R) * D
            return [pltpu.async_copy(
                ob[p].at[pl.ds(b * W, W)],
                out_hbm.at[pl.ds(b * L * D + lbase, W)], ssem[p])
                for b in range(B)]

        loads = {0: start_load(0), 1: start_load(1)}
        stores = {}
        for c in range(CH):
            p = c % 2
            for h in loads.pop(c):
                h.wait()
            if c >= 2:
                for h in stores.pop(c - 2):
                    h.wait()

            def add_body(i, carry, p=p):
                off = i * (U * 16)
                for u in range(U):
                    o = off + u * 16
                    pv = pb[p][pl.ds(o, 16)]
                    for b in range(B):
                        bo = b * W + o
                        ob[p][pl.ds(bo, 16)] = xb[p][pl.ds(bo, 16)] + pv
                return carry

            lax.fori_loop(0, NV // U, add_body, 0)
            stores[c] = start_store(c)
            if c + 2 < CH:
                loads[c + 2] = start_load(c + 2)
        for c in sorted(stores):
            for h in stores[c]:
                h.wait()

    return k


def _sc_kernel_v2(x, pos_table):
    B, L, D = x.shape
    k = _make_sc_kernel_v2(B, L, D)
    out = k(x.reshape(-1), pos_table[:L].reshape(-1))
    return out.reshape(B, L, D)


def _make_sc_kernel_v3(B, L, D):
    l_per_w = L // _NW
    R = 4
    CH = l_per_w // R
    W = R * D
    NV = W // 16
    U = 4
    mesh = plsc.VectorSubcoreMesh(core_axis_name="c", subcore_axis_name="s")

    @functools.partial(
        pl.kernel,
        mesh=mesh,
        out_type=jax.ShapeDtypeStruct((B, L * D), jnp.float32),
        scratch_types=[
            pltpu.VMEM((B, W), jnp.float32), pltpu.VMEM((B, W), jnp.float32),
            pltpu.VMEM((W,), jnp.float32), pltpu.VMEM((W,), jnp.float32),
            pltpu.VMEM((B, W), jnp.float32), pltpu.VMEM((B, W), jnp.float32),
            pltpu.SemaphoreType.DMA, pltpu.SemaphoreType.DMA,
            pltpu.SemaphoreType.DMA, pltpu.SemaphoreType.DMA,
        ],
    )
    def k(x_hbm, pos_hbm, out_hbm, xb0, xb1, pb0, pb1, ob0, ob1,
          ls0, ls1, ss0, ss1):
        xb, pb, ob = (xb0, xb1), (pb0, pb1), (ob0, ob1)
        lsem, ssem = (ls0, ls1), (ss0, ss1)
        wid = lax.axis_index("s") * _NC + lax.axis_index("c")
        l0 = wid * l_per_w

        def start_load(c):
            p = c % 2
            lbase = (l0 + c * R) * D
            return [
                pltpu.async_copy(pos_hbm.at[pl.ds(lbase, W)], pb[p], lsem[p]),
                pltpu.async_copy(x_hbm.at[:, pl.ds(lbase, W)], xb[p], lsem[p]),
            ]

        def start_store(c):
            p = c % 2
            lbase = (l0 + c * R) * D
            return [pltpu.async_copy(ob[p], out_hbm.at[:, pl.ds(lbase, W)],
                                     ssem[p])]

        loads = {0: start_load(0), 1: start_load(1)}
        stores = {}
        for c in range(CH):
            p = c % 2
            for h in loads.pop(c):
                h.wait()
            if c >= 2:
                for h in stores.pop(c - 2):
                    h.wait()

            def add_body(i, carry, p=p):
                off = i * (U * 16)
                for u in range(U):
                    o = off + u * 16
                    pv = pb[p][pl.ds(o, 16)]
                    for b in range(B):
                        ob[p][b, pl.ds(o, 16)] = xb[p][b, pl.ds(o, 16)] + pv
                return carry

            lax.fori_loop(0, NV // U, add_body, 0)
            stores[c] = start_store(c)
            if c + 2 < CH:
                loads[c + 2] = start_load(c + 2)
        for c in sorted(stores):
            for h in stores[c]:
                h.wait()

    return k


def _sc_kernel_v3(x, pos_table):
    B, L, D = x.shape
    k = _make_sc_kernel_v3(B, L, D)
    out = k(x.reshape(B, L * D), pos_table[:L].reshape(-1))
    return out.reshape(B, L, D)


def kernel(x, pos_table):
    return _sc_kernel_v3(x, pos_table)


# speedup vs baseline: 6.0696x; 6.0696x over previous
import jax
import jax.numpy as jnp
from jax.experimental import pallas as pl


def _body(x_ref, p_ref, o_ref):
    o_ref[...] = x_ref[...] + p_ref[...]


def kernel(x, pos_table):
    B, L, D = x.shape
    BL = 2048
    grid = (L // BL, B)
    return pl.pallas_call(
        _body,
        grid=grid,
        in_specs=[
            pl.BlockSpec((1, BL, D), lambda l, b: (b, l, 0)),
            pl.BlockSpec((BL, D), lambda l, b: (l, 0)),
        ],
        out_specs=pl.BlockSpec((1, BL, D), lambda l, b: (b, l, 0)),
        out_shape=jax.ShapeDtypeStruct((B, L, D), x.dtype),
    )(x, pos_table)
